# baseline (speedup 1.0000x reference)
import functools

import jax
import jax.numpy as jnp
from jax import lax
from jax.experimental import pallas as pl
from jax.experimental.pallas import tpu as pltpu
from jax.experimental.pallas import tpu_sc as plsc

G_SEGS = 10000
C = 50
CP = 64

NW = 32
CHUNK = 128
STG = 8
USTG = 4

N2P = 229376
N34P = 425984

G_ACC = 10240

TB = 2048


def _harmonic_body(n_rows, x_ref, k_ref, eq_ref, o_ref):
    i = pl.program_id(0)
    rows = i * TB + lax.broadcasted_iota(jnp.int32, (TB, 1), 0)
    valid = rows < n_rows
    x = jnp.where(valid, x_ref[...], 0.0)
    k = jnp.where(valid, k_ref[...], 0.0)
    eq = jnp.where(valid, eq_ref[...], 0.0)
    d = x - eq
    u = 0.5 * k * d * d
    o_ref[...] = jnp.pad(u, ((0, 0), (0, CP - C)))


def _harmonic_u(x, k, eq, n_pad):
    n = x.shape[0]
    grid = n_pad // TB
    last = (n - 1) // TB
    imap = lambda i: (jnp.minimum(i, last), 0)
    return pl.pallas_call(
        functools.partial(_harmonic_body, n),
        grid=(grid,),
        in_specs=[
            pl.BlockSpec((TB, C), imap),
            pl.BlockSpec((TB, 1), imap),
            pl.BlockSpec((TB, 1), imap),
        ],
        out_specs=pl.BlockSpec((TB, CP), lambda i: (i, 0)),
        out_shape=jax.ShapeDtypeStruct((n_pad, CP), jnp.float32),
        compiler_params=pltpu.CompilerParams(
            dimension_semantics=("arbitrary",)),
    )(x, k, eq)


def _torsion_body(n_rows, x_ref, k_ref, ph_ref, pn_ref, o_ref):
    i = pl.program_id(0)
    rows = i * TB + lax.broadcasted_iota(jnp.int32, (TB, 1), 0)
    valid = rows < n_rows
    x = jnp.where(valid, x_ref[...], 0.0)
    k = jnp.where(valid, k_ref[...], 0.0)
    ph = jnp.where(valid, ph_ref[...], 0.0)
    pn = jnp.where(valid, pn_ref[...].astype(jnp.float32), 0.0)
    u = jnp.zeros((TB, C), jnp.float32)
    for j in range(6):
        u = u + k[:, j:j + 1] * (
            1.0 + jnp.cos(pn[:, j:j + 1] * x - ph[:, j:j + 1]))
    o_ref[...] = jnp.pad(u, ((0, 0), (0, CP - C)))


def _torsion_u(x, k, ph, pn, n_pad):
    n = x.shape[0]
    grid = n_pad // TB
    last = (n - 1) // TB
    imap = lambda i: (jnp.minimum(i, last), 0)
    return pl.pallas_call(
        functools.partial(_torsion_body, n),
        grid=(grid,),
        in_specs=[
            pl.BlockSpec((TB, C), imap),
            pl.BlockSpec((TB, 6), imap),
            pl.BlockSpec((TB, 6), imap),
            pl.BlockSpec((TB, 6), imap),
        ],
        out_specs=pl.BlockSpec((TB, CP), lambda i: (i, 0)),
        out_shape=jax.ShapeDtypeStruct((n_pad, CP), jnp.float32),
        compiler_params=pltpu.CompilerParams(
            dimension_semantics=("arbitrary",)),
    )(x, k, ph, pn)


_ROWS_PER_TILE_OUT = G_ACC // 16


def _sc_scatter(u2, idx2, u3, idx3, u4, idx4, zrows):
    mesh = plsc.VectorSubcoreMesh(core_axis_name="c", subcore_axis_name="s",
                                  num_cores=2, num_subcores=16)

    @functools.partial(
        pl.kernel,
        out_type=jax.ShapeDtypeStruct((2 * G_ACC, CP), jnp.float32),
        mesh=mesh,
        compiler_params=pltpu.CompilerParams(use_tc_tiling_on_sc=False),
        scratch_types=[
            pltpu.VMEM((USTG * CHUNK, CP), jnp.float32),
            pltpu.VMEM((STG, CHUNK), jnp.int32),
            pltpu.VMEM((_ROWS_PER_TILE_OUT // 4, CP), jnp.float32),
            pltpu.VMEM_SHARED((G_ACC, CP), jnp.float32),
        ],
    )
    def k(u2_hbm, idx2_hbm, u3_hbm, idx3_hbm, u4_hbm, idx4_hbm, z_hbm,
          out_hbm, ubuf, ibuf, obuf, acc):
        cid = lax.axis_index("c")
        sid = lax.axis_index("s")
        w = cid * 16 + sid

        quarter = _ROWS_PER_TILE_OUT // 4
        pltpu.sync_copy(z_hbm, obuf)
        for hh in range(4):
            z0 = pl.multiple_of(sid * _ROWS_PER_TILE_OUT + hh * quarter,
                                quarter)
            pltpu.sync_copy(obuf, acc.at[pl.ds(z0, quarter)])
        plsc.subcore_barrier()

        for u_hbm, idx_hbm, n_pad in ((u2_hbm, idx2_hbm, N2P),
                                      (u3_hbm, idx3_hbm, N34P),
                                      (u4_hbm, idx4_hbm, N34P)):
            rows_per_tile = n_pad // NW
            n_outer = rows_per_tile // (STG * CHUNK)
            row0 = w * rows_per_tile
            chunk0 = row0 // CHUNK

            def body(it, _, u_hbm=u_hbm, idx_hbm=idx_hbm, row0=row0,
                     chunk0=chunk0):
                ioff = pl.multiple_of(chunk0 + it * STG, STG)
                pltpu.sync_copy(idx_hbm.at[pl.ds(ioff, STG)], ibuf)
                for h in range(STG // USTG):
                    uoff = pl.multiple_of(
                        row0 + it * (STG * CHUNK) + h * (USTG * CHUNK),
                        USTG * CHUNK)
                    pltpu.sync_copy(u_hbm.at[pl.ds(uoff, USTG * CHUNK)], ubuf)
                    for j in range(USTG):
                        pltpu.sync_copy(ubuf.at[pl.ds(j * CHUNK, CHUNK)],
                                        acc.at[ibuf.at[h * USTG + j]],
                                        add=True)
                return ()

            lax.fori_loop(0, n_outer, body, ())

        plsc.subcore_barrier()
        half = _ROWS_PER_TILE_OUT // 4
        for hh in range(4):
            r0 = pl.multiple_of(sid * _ROWS_PER_TILE_OUT + hh * half, half)
            pltpu.sync_copy(acc.at[pl.ds(r0, half)], obuf)
            pltpu.sync_copy(obuf,
                            out_hbm.at[pl.ds(pl.multiple_of(
                                cid * G_ACC + r0, half), half)])

    return k(u2, idx2, u3, idx3, u4, idx4, zrows)


def _combine_body(a_ref, b_ref, o_ref):
    o_ref[...] = (a_ref[...] + b_ref[...])[:, :C]


def _combine(p1, p2):
    gb = 1000
    return pl.pallas_call(
        _combine_body,
        grid=(G_SEGS // gb,),
        in_specs=[
            pl.BlockSpec((gb, CP), lambda i: (i, 0)),
            pl.BlockSpec((gb, CP), lambda i: (i, 0)),
        ],
        out_specs=pl.BlockSpec((gb, C), lambda i: (i, 0)),
        out_shape=jax.ShapeDtypeStruct((G_SEGS, C), jnp.float32),
    )(p1, p2)


def _pad_idx(idx, n_pad):
    n = idx.shape[0]
    return jnp.pad(idx, (0, n_pad - n)).reshape(n_pad // CHUNK, CHUNK)


def kernel(x2, k2, eq2, idx2, x3, k3, eq3, idx3, x4, k4, phases, periodicity,
           idx4):
    u2 = _harmonic_u(x2, k2, eq2, N2P)
    u3 = _harmonic_u(x3, k3, eq3, N34P)
    u4 = _torsion_u(x4, k4, phases, periodicity, N34P)
    zrows = jnp.zeros((_ROWS_PER_TILE_OUT // 4, CP), jnp.float32)
    pp = _sc_scatter(u2, _pad_idx(idx2, N2P),
                     u3, _pad_idx(idx3, N34P),
                     u4, _pad_idx(idx4, N34P), zrows)
    return _combine(pp[:G_SEGS], pp[G_ACC:G_ACC + G_SEGS])
